# baseline (speedup 1.0000x reference)
import functools

import jax
import jax.numpy as jnp
from jax.experimental import pallas as pl
from jax.experimental.pallas import tpu as pltpu

NUM_EMB = 1024
DIM = 64
B = 8
H = 32
W = 32
HW = H * W
COMMIT = 0.25


def _vq_body(x_ref, e_ref, q_ref, idx_ref, loss_ref):
    x = jnp.concatenate([x_ref[0, :, i, :] for i in range(H)], axis=1)
    e = e_ref[...]
    enorm = jnp.sum(e * e, axis=1, keepdims=True)
    xnorm = jnp.sum(x * x, axis=0, keepdims=True)
    mm2 = jax.lax.dot_general(e + e, x, (((1,), (0,)), ((), ())),
                              preferred_element_type=jnp.float32)
    d = (xnorm + enorm) - mm2
    vmin = jnp.min(d, axis=0, keepdims=True)
    iota = jax.lax.broadcasted_iota(jnp.int32, (NUM_EMB, HW), 0)
    idx = jnp.min(jnp.where(d == vmin, iota, NUM_EMB), axis=0, keepdims=True)
    idx = idx.astype(jnp.int32)
    for i in range(H):
        idx_ref[0, i:i + 1, :] = idx[:, i * W:(i + 1) * W]
    onehot = (iota == idx).astype(jnp.float32)
    q = jax.lax.dot_general(e, onehot, (((0,), (0,)), ((), ())),
                            preferred_element_type=jnp.float32)
    for i in range(H):
        q_ref[0, :, i, :] = q[:, i * W:(i + 1) * W]
    b = pl.program_id(0)
    r = q - x
    part = jnp.sum(r * r, keepdims=True).reshape(1, 1)

    @pl.when(b == 0)
    def _():
        loss_ref[...] = jnp.zeros((1, 1), jnp.float32)

    acc = loss_ref[...] + part
    loss_ref[...] = jnp.where(b == B - 1,
                              acc * ((1.0 + COMMIT) / (B * HW * DIM)), acc)


@functools.partial(jax.jit, static_argnames=())
def kernel(inputs, embedding_weight):
    q, idx, loss = pl.pallas_call(
        _vq_body,
        grid=(B,),
        in_specs=[
            pl.BlockSpec((1, DIM, H, W), lambda b: (b, 0, 0, 0)),
            pl.BlockSpec((NUM_EMB, DIM), lambda b: (0, 0)),
        ],
        out_specs=[
            pl.BlockSpec((1, DIM, H, W), lambda b: (b, 0, 0, 0)),
            pl.BlockSpec((1, H, W), lambda b: (b, 0, 0)),
            pl.BlockSpec((1, 1), lambda b: (0, 0)),
        ],
        out_shape=[
            jax.ShapeDtypeStruct((B, DIM, H, W), jnp.float32),
            jax.ShapeDtypeStruct((B, H, W), jnp.int32),
            jax.ShapeDtypeStruct((1, 1), jnp.float32),
        ],
    )(inputs, embedding_weight)
    return q, loss[0, 0], idx


# speedup vs baseline: 2.2300x; 2.2300x over previous
import functools

import jax
import jax.numpy as jnp
from jax.experimental import pallas as pl
from jax.experimental.pallas import tpu as pltpu

NUM_EMB = 1024
DIM = 64
B = 8
HW = 1024
COMMIT = 0.25
G = NUM_EMB // 8


def _vq_body(x_ref, e_ref, q_ref, idx_ref, loss_ref):
    x = x_ref[0]
    e = e_ref[...]
    enorm = jnp.sum(e * e, axis=1, keepdims=True)
    xnorm = jnp.sum(x * x, axis=0, keepdims=True)
    mm2 = jax.lax.dot_general(e + e, x, (((1,), (0,)), ((), ())),
                              preferred_element_type=jnp.float32)
    d = (xnorm + enorm) - mm2
    d3 = d.reshape(G, 8, HW)
    vals = d3[0]
    gwin = jnp.zeros((8, HW), jnp.int32)
    for g in range(1, G):
        dg = d3[g]
        lt = dg < vals
        vals = jnp.where(lt, dg, vals)
        gwin = jnp.where(lt, g, gwin)
    vmin = jnp.min(vals, axis=0, keepdims=True)
    siota = jax.lax.broadcasted_iota(jnp.int32, (8, HW), 0)
    j8 = gwin * 8 + siota
    idx = jnp.min(jnp.where(vals == vmin, j8, NUM_EMB), axis=0, keepdims=True)
    idx = idx.astype(jnp.int32)
    idx_ref[0] = idx
    iota = jax.lax.broadcasted_iota(jnp.int32, (NUM_EMB, HW), 0)
    onehot = (iota == idx).astype(jnp.float32)
    q = jax.lax.dot_general(e, onehot, (((0,), (0,)), ((), ())),
                            preferred_element_type=jnp.float32)
    q_ref[0] = q
    b = pl.program_id(0)
    r = q - x
    part = jnp.sum(r * r, keepdims=True).reshape(1, 1)

    @pl.when(b == 0)
    def _():
        loss_ref[...] = jnp.zeros((1, 1), jnp.float32)

    acc = loss_ref[...] + part
    loss_ref[...] = jnp.where(b == B - 1,
                              acc * ((1.0 + COMMIT) / (B * HW * DIM)), acc)


@functools.partial(jax.jit, static_argnames=())
def kernel(inputs, embedding_weight):
    x = inputs.reshape(B, DIM, HW)
    q, idx, loss = pl.pallas_call(
        _vq_body,
        grid=(B,),
        in_specs=[
            pl.BlockSpec((1, DIM, HW), lambda b: (b, 0, 0)),
            pl.BlockSpec((NUM_EMB, DIM), lambda b: (0, 0)),
        ],
        out_specs=[
            pl.BlockSpec((1, DIM, HW), lambda b: (b, 0, 0)),
            pl.BlockSpec((1, 1, HW), lambda b: (b, 0, 0)),
            pl.BlockSpec((1, 1), lambda b: (0, 0)),
        ],
        out_shape=[
            jax.ShapeDtypeStruct((B, DIM, HW), jnp.float32),
            jax.ShapeDtypeStruct((B, 1, HW), jnp.int32),
            jax.ShapeDtypeStruct((1, 1), jnp.float32),
        ],
    )(x, embedding_weight)
    quantized_st = q.reshape(inputs.shape)
    vq_loss = loss[0, 0]
    indices = idx.reshape(B, 32, 32)
    return quantized_st, vq_loss, indices


# speedup vs baseline: 2.2335x; 1.0016x over previous
import functools

import jax
import jax.numpy as jnp
from jax.experimental import pallas as pl
from jax.experimental.pallas import tpu as pltpu

NUM_EMB = 1024
DIM = 64
B = 8
HW = 1024
COMMIT = 0.25
G = NUM_EMB // 8


def _vq_body(x_ref, e_ref, q_ref, idx_ref, loss_ref):
    x = x_ref[0]
    e = e_ref[...]
    enorm = jnp.sum(e * e, axis=1, keepdims=True)
    xnorm = jnp.sum(x * x, axis=0, keepdims=True)
    mm2 = jax.lax.dot_general(e + e, x, (((1,), (0,)), ((), ())),
                              preferred_element_type=jnp.float32)
    mm3 = mm2.reshape(G, 8, HW)
    en3 = enorm.reshape(G, 8, 1)
    vals = (xnorm + en3[0]) - mm3[0]
    gwin = jnp.zeros((8, HW), jnp.int32)
    for g in range(1, G):
        dg = (xnorm + en3[g]) - mm3[g]
        lt = dg < vals
        vals = jnp.where(lt, dg, vals)
        gwin = jnp.where(lt, g, gwin)
    vmin = jnp.min(vals, axis=0, keepdims=True)
    siota = jax.lax.broadcasted_iota(jnp.int32, (8, HW), 0)
    j8 = gwin * 8 + siota
    idx = jnp.min(jnp.where(vals == vmin, j8, NUM_EMB), axis=0, keepdims=True)
    idx = idx.astype(jnp.int32)
    idx_ref[0] = idx
    iota = jax.lax.broadcasted_iota(jnp.int32, (NUM_EMB, HW), 0)
    onehot = (iota == idx).astype(jnp.float32)
    q = jax.lax.dot_general(e, onehot, (((0,), (0,)), ((), ())),
                            preferred_element_type=jnp.float32)
    q_ref[0] = q
    b = pl.program_id(0)
    r = q - x
    part = jnp.sum(r * r, keepdims=True).reshape(1, 1)

    @pl.when(b == 0)
    def _():
        loss_ref[...] = jnp.zeros((1, 1), jnp.float32)

    acc = loss_ref[...] + part
    loss_ref[...] = jnp.where(b == B - 1,
                              acc * ((1.0 + COMMIT) / (B * HW * DIM)), acc)


@functools.partial(jax.jit, static_argnames=())
def kernel(inputs, embedding_weight):
    x = inputs.reshape(B, DIM, HW)
    q, idx, loss = pl.pallas_call(
        _vq_body,
        grid=(B,),
        in_specs=[
            pl.BlockSpec((1, DIM, HW), lambda b: (b, 0, 0)),
            pl.BlockSpec((NUM_EMB, DIM), lambda b: (0, 0)),
        ],
        out_specs=[
            pl.BlockSpec((1, DIM, HW), lambda b: (b, 0, 0)),
            pl.BlockSpec((1, 1, HW), lambda b: (b, 0, 0)),
            pl.BlockSpec((1, 1), lambda b: (0, 0)),
        ],
        out_shape=[
            jax.ShapeDtypeStruct((B, DIM, HW), jnp.float32),
            jax.ShapeDtypeStruct((B, 1, HW), jnp.int32),
            jax.ShapeDtypeStruct((1, 1), jnp.float32),
        ],
    )(x, embedding_weight)
    quantized_st = q.reshape(inputs.shape)
    vq_loss = loss[0, 0]
    indices = idx.reshape(B, 32, 32)
    return quantized_st, vq_loss, indices


# speedup vs baseline: 2.3729x; 1.0624x over previous
import functools

import jax
import jax.numpy as jnp
from jax.experimental import pallas as pl
from jax.experimental.pallas import tpu as pltpu

NUM_EMB = 1024
DIM = 64
B = 8
HW = 1024
COMMIT = 0.25
G = NUM_EMB // 8


BLK = 2
P = BLK * HW


def _vq_body(x_ref, e_ref, q_ref, idx_ref, loss_ref):
    x = jnp.concatenate([x_ref[i] for i in range(BLK)], axis=1)
    e = e_ref[...]
    enorm = jnp.sum(e * e, axis=1, keepdims=True)
    xnorm = jnp.sum(x * x, axis=0, keepdims=True)
    mm2 = jax.lax.dot_general(e + e, x, (((1,), (0,)), ((), ())),
                              preferred_element_type=jnp.float32)
    mm3 = mm2.reshape(G, 8, P)
    en3 = enorm.reshape(G, 8, 1)
    vals = (xnorm + en3[0]) - mm3[0]
    gwin = jnp.zeros((8, P), jnp.int32)
    for g in range(1, G):
        dg = (xnorm + en3[g]) - mm3[g]
        lt = dg < vals
        vals = jnp.where(lt, dg, vals)
        gwin = jnp.where(lt, g, gwin)
    vmin = jnp.min(vals, axis=0, keepdims=True)
    siota = jax.lax.broadcasted_iota(jnp.int32, (8, P), 0)
    j8 = gwin * 8 + siota
    idx = jnp.min(jnp.where(vals == vmin, j8, NUM_EMB), axis=0, keepdims=True)
    idx = idx.astype(jnp.int32)
    for i in range(BLK):
        idx_ref[i] = idx[:, i * HW:(i + 1) * HW]
    iota = jax.lax.broadcasted_iota(jnp.int32, (NUM_EMB, P), 0)
    onehot = (iota == idx).astype(jnp.float32)
    q = jax.lax.dot_general(e, onehot, (((0,), (0,)), ((), ())),
                            preferred_element_type=jnp.float32)
    for i in range(BLK):
        q_ref[i] = q[:, i * HW:(i + 1) * HW]
    b = pl.program_id(0)
    r = q - x
    part = jnp.sum(r * r, keepdims=True).reshape(1, 1)

    @pl.when(b == 0)
    def _():
        loss_ref[...] = jnp.zeros((1, 1), jnp.float32)

    acc = loss_ref[...] + part
    loss_ref[...] = jnp.where(b == B // BLK - 1,
                              acc * ((1.0 + COMMIT) / (B * HW * DIM)), acc)


@functools.partial(jax.jit, static_argnames=())
def kernel(inputs, embedding_weight):
    x = inputs.reshape(B, DIM, HW)
    q, idx, loss = pl.pallas_call(
        _vq_body,
        grid=(B // BLK,),
        in_specs=[
            pl.BlockSpec((BLK, DIM, HW), lambda b: (b, 0, 0)),
            pl.BlockSpec((NUM_EMB, DIM), lambda b: (0, 0)),
        ],
        out_specs=[
            pl.BlockSpec((BLK, DIM, HW), lambda b: (b, 0, 0)),
            pl.BlockSpec((BLK, 1, HW), lambda b: (b, 0, 0)),
            pl.BlockSpec((1, 1), lambda b: (0, 0)),
        ],
        out_shape=[
            jax.ShapeDtypeStruct((B, DIM, HW), jnp.float32),
            jax.ShapeDtypeStruct((B, 1, HW), jnp.int32),
            jax.ShapeDtypeStruct((1, 1), jnp.float32),
        ],
    )(x, embedding_weight)
    quantized_st = q.reshape(inputs.shape)
    vq_loss = loss[0, 0]
    indices = idx.reshape(B, 32, 32)
    return quantized_st, vq_loss, indices


# speedup vs baseline: 2.3784x; 1.0023x over previous
import functools

import jax
import jax.numpy as jnp
from jax.experimental import pallas as pl
from jax.experimental.pallas import tpu as pltpu

NUM_EMB = 1024
DIM = 64
B = 8
HW = 1024
COMMIT = 0.25
G = NUM_EMB // 8


BLK = 4
P = BLK * HW


def _vq_body(x_ref, e_ref, q_ref, idx_ref, loss_ref):
    x = jnp.concatenate([x_ref[i] for i in range(BLK)], axis=1)
    e = e_ref[...]
    enorm = jnp.sum(e * e, axis=1, keepdims=True)
    xnorm = jnp.sum(x * x, axis=0, keepdims=True)
    mm2 = jax.lax.dot_general(e + e, x, (((1,), (0,)), ((), ())),
                              preferred_element_type=jnp.float32)
    mm3 = mm2.reshape(G, 8, P)
    en3 = enorm.reshape(G, 8, 1)
    vals = (xnorm + en3[0]) - mm3[0]
    gwin = jnp.zeros((8, P), jnp.int32)
    for g in range(1, G):
        dg = (xnorm + en3[g]) - mm3[g]
        lt = dg < vals
        vals = jnp.where(lt, dg, vals)
        gwin = jnp.where(lt, g, gwin)
    vmin = jnp.min(vals, axis=0, keepdims=True)
    siota = jax.lax.broadcasted_iota(jnp.int32, (8, P), 0)
    j8 = gwin * 8 + siota
    idx = jnp.min(jnp.where(vals == vmin, j8, NUM_EMB), axis=0, keepdims=True)
    idx = idx.astype(jnp.int32)
    for i in range(BLK):
        idx_ref[i] = idx[:, i * HW:(i + 1) * HW]
    iota = jax.lax.broadcasted_iota(jnp.int32, (NUM_EMB, P), 0)
    onehot = (iota == idx).astype(jnp.float32)
    q = jax.lax.dot_general(e, onehot, (((0,), (0,)), ((), ())),
                            preferred_element_type=jnp.float32)
    for i in range(BLK):
        q_ref[i] = q[:, i * HW:(i + 1) * HW]
    b = pl.program_id(0)
    r = q - x
    part = jnp.sum(r * r, keepdims=True).reshape(1, 1)

    @pl.when(b == 0)
    def _():
        loss_ref[...] = jnp.zeros((1, 1), jnp.float32)

    acc = loss_ref[...] + part
    loss_ref[...] = jnp.where(b == B // BLK - 1,
                              acc * ((1.0 + COMMIT) / (B * HW * DIM)), acc)


@functools.partial(jax.jit, static_argnames=())
def kernel(inputs, embedding_weight):
    x = inputs.reshape(B, DIM, HW)
    q, idx, loss = pl.pallas_call(
        _vq_body,
        grid=(B // BLK,),
        in_specs=[
            pl.BlockSpec((BLK, DIM, HW), lambda b: (b, 0, 0)),
            pl.BlockSpec((NUM_EMB, DIM), lambda b: (0, 0)),
        ],
        out_specs=[
            pl.BlockSpec((BLK, DIM, HW), lambda b: (b, 0, 0)),
            pl.BlockSpec((BLK, 1, HW), lambda b: (b, 0, 0)),
            pl.BlockSpec((1, 1), lambda b: (0, 0)),
        ],
        out_shape=[
            jax.ShapeDtypeStruct((B, DIM, HW), jnp.float32),
            jax.ShapeDtypeStruct((B, 1, HW), jnp.int32),
            jax.ShapeDtypeStruct((1, 1), jnp.float32),
        ],
    )(x, embedding_weight)
    quantized_st = q.reshape(inputs.shape)
    vq_loss = loss[0, 0]
    indices = idx.reshape(B, 32, 32)
    return quantized_st, vq_loss, indices
